# baseline (speedup 1.0000x reference)
import functools

import jax
import jax.numpy as jnp
from jax import lax
from jax.experimental import pallas as pl
from jax.experimental.pallas import tpu as pltpu
from jax.experimental.pallas import tpu_sc as plsc

N_NODES = 10000
NODE_DIM = 128
EDGE_DIM = 16
HIDDEN_DIM = 128
N_EDGES = 320000

NC = 2
NS = 16
NW = NC * NS
PHASE_EPW = (1000, 4000, 4000, 1000)
P = len(PHASE_EPW)
CH = 80
STRIPE = 1000
KG = 6
KS = 4


def _split(epw, per_iter):
    outer = epw // per_iter
    rem = epw - outer * per_iter
    tail = []
    while rem >= CH:
        tail.append(CH)
        rem -= CH
    if rem:
        assert rem % 8 == 0
        tail.append(rem)
    return outer, tuple(tail)


@functools.cache
def _make_sc_gather(epw):
    outer, tail = _split(epw, KG * CH)
    mesh = plsc.VectorSubcoreMesh(core_axis_name="c", subcore_axis_name="s")

    def body_fn(x_hbm, src_hbm, out_hbm, idx_v, rows_v, sem):
        wid = lax.axis_index("s") * NC + lax.axis_index("c")

        def step(base, chunks):
            n = sum(chunks)
            pltpu.sync_copy(src_hbm.at[pl.ds(base, n)], idx_v.at[pl.ds(0, n)])
            cps, off = [], 0
            for c in chunks:
                cps.append(
                    pltpu.async_copy(
                        x_hbm.at[idx_v.at[pl.ds(off, c)]],
                        rows_v.at[pl.ds(off, c)],
                        sem,
                    )
                )
                off += c
            for cp in cps:
                cp.wait()
            pltpu.sync_copy(rows_v.at[pl.ds(0, n)], out_hbm.at[pl.ds(base, n)])

        def body(j, carry):
            step(wid * epw + j * (KG * CH), (CH,) * KG)
            return carry

        lax.fori_loop(0, outer, body, 0)
        if tail:
            step(wid * epw + outer * KG * CH, tail)

    return pl.kernel(
        body_fn,
        mesh=mesh,
        out_type=jax.ShapeDtypeStruct((epw * NW, NODE_DIM), jnp.float32),
        scratch_types=[
            pltpu.VMEM((KG * CH,), jnp.int32),
            pltpu.VMEM((KG * CH, NODE_DIM), jnp.float32),
            pltpu.SemaphoreType.DMA,
        ],
    )


@functools.cache
def _make_sc_scatter(epw):
    outer, tail = _split(epw, KS * CH)
    mesh = plsc.VectorSubcoreMesh(core_axis_name="c", subcore_axis_name="s")

    def body_fn(msg_hbm, dst_hbm, zero_hbm, out_hbm, idx_vs, idx_t, rows_v, agg_sh, isem):
        cid = lax.axis_index("c")
        sid = lax.axis_index("s")
        wid = sid * NC + cid
        @pl.when(sid < N_NODES // STRIPE)
        def _():
            pltpu.sync_copy(
                zero_hbm.at[pl.ds(sid * STRIPE, STRIPE)],
                agg_sh.at[pl.ds(sid * STRIPE, STRIPE)],
            )

        plsc.subcore_barrier()

        def step(base, bufs):
            n = sum(b.shape[0] for b in bufs)
            icps, off = [], 0
            for b in bufs:
                icps.append(
                    pltpu.async_copy(
                        dst_hbm.at[pl.ds(base + off, b.shape[0])], b, isem
                    )
                )
                off += b.shape[0]
            pltpu.sync_copy(msg_hbm.at[pl.ds(base, n)], rows_v.at[pl.ds(0, n)])
            for cp in icps:
                cp.wait()
            off = 0
            for b in bufs:
                pltpu.sync_copy(
                    rows_v.at[pl.ds(off, b.shape[0])], agg_sh.at[b], add=True
                )
                off += b.shape[0]

        def body(j, carry):
            step(wid * epw + j * (KS * CH), idx_vs)
            return carry

        lax.fori_loop(0, outer, body, 0)
        if tail:
            tail_bufs = [
                idx_vs[i] if c == CH else idx_t for i, c in enumerate(tail)
            ]
            step(wid * epw + outer * (KS * CH), tail_bufs)
        plsc.subcore_barrier()

        @pl.when(sid < N_NODES // STRIPE)
        def _():
            pltpu.sync_copy(
                agg_sh.at[pl.ds(sid * STRIPE, STRIPE)],
                out_hbm.at[pl.ds(cid * N_NODES + sid * STRIPE, STRIPE)],
            )

    return pl.kernel(
        body_fn,
        mesh=mesh,
        out_type=jax.ShapeDtypeStruct((NC * N_NODES, HIDDEN_DIM), jnp.float32),
        scratch_types=[
            [pltpu.VMEM((CH,), jnp.int32) for _ in range(KS)],
            pltpu.VMEM((40,), jnp.int32),
            pltpu.VMEM((KS * CH, HIDDEN_DIM), jnp.float32),
            pltpu.VMEM_SHARED((N_NODES, HIDDEN_DIM), jnp.float32),
            pltpu.SemaphoreType.DMA,
        ],
    )


def _mlp_body(gx_ref, ea_ref, w1x_ref, w1e_ref, b1_ref, w2_ref, b2_ref, out_ref):
    gx = gx_ref[...].astype(jnp.bfloat16)
    ea_t = ea_ref[...].astype(jnp.bfloat16)
    h = jnp.dot(gx, w1x_ref[...], preferred_element_type=jnp.float32)
    h = h + lax.dot_general(
        ea_t,
        w1e_ref[...],
        dimension_numbers=(((0,), (0,)), ((), ())),
        preferred_element_type=jnp.float32,
    )
    h = jnp.maximum(h + b1_ref[...], 0.0).astype(jnp.bfloat16)
    out_ref[...] = (
        jnp.dot(h, w2_ref[...], preferred_element_type=jnp.float32) + b2_ref[...]
    )


def _gru_body(*refs):
    part_refs = refs[: 2 * P]
    x_ref, wih_ref, whh_ref, bih_ref, bhh_ref, out_ref = refs[2 * P :]
    agg = part_refs[0][...]
    for r in part_refs[1:]:
        agg = agg + r[...]
    x = x_ref[...]
    gi = jnp.dot(agg, wih_ref[...], preferred_element_type=jnp.float32) + bih_ref[...]
    gh = jnp.dot(x, whh_ref[...], preferred_element_type=jnp.float32) + bhh_ref[...]
    i_r = gi[:, :NODE_DIM]
    i_z = gi[:, NODE_DIM : 2 * NODE_DIM]
    i_n = gi[:, 2 * NODE_DIM :]
    h_r = gh[:, :NODE_DIM]
    h_z = gh[:, NODE_DIM : 2 * NODE_DIM]
    h_n = gh[:, 2 * NODE_DIM :]
    r = jax.nn.sigmoid(i_r + h_r)
    z = jax.nn.sigmoid(i_z + h_z)
    n = jnp.tanh(i_n + r * h_n)
    out_ref[...] = (1.0 - z) * n + z * x


BE = 6400
BN = 2000


def kernel(x, edge_index, edge_attr, W1, b1, W2, b2, W_ih, b_ih, W_hh, b_hh):
    src = edge_index[0].astype(jnp.int32)
    dst = edge_index[1].astype(jnp.int32)
    ea_t = edge_attr.T

    w1x_t = W1[:, :NODE_DIM].T.astype(jnp.bfloat16)
    w1e_t = W1[:, NODE_DIM:].T.astype(jnp.bfloat16)
    w2_t = W2.T.astype(jnp.bfloat16)
    zero = jnp.zeros((N_NODES, HIDDEN_DIM), jnp.float32)

    def mlp(gathered, edge_base, n_edges):
        return pl.pallas_call(
            _mlp_body,
            grid=(n_edges // BE,),
            in_specs=[
                pl.BlockSpec((BE, NODE_DIM), lambda i: (i, 0)),
                pl.BlockSpec(
                    (EDGE_DIM, BE), lambda i, b=edge_base // BE: (0, b + i)
                ),
                pl.BlockSpec((NODE_DIM, HIDDEN_DIM), lambda i: (0, 0)),
                pl.BlockSpec((EDGE_DIM, HIDDEN_DIM), lambda i: (0, 0)),
                pl.BlockSpec((1, HIDDEN_DIM), lambda i: (0, 0)),
                pl.BlockSpec((HIDDEN_DIM, HIDDEN_DIM), lambda i: (0, 0)),
                pl.BlockSpec((1, HIDDEN_DIM), lambda i: (0, 0)),
            ],
            out_specs=pl.BlockSpec((BE, HIDDEN_DIM), lambda i: (i, 0)),
            out_shape=jax.ShapeDtypeStruct((n_edges, HIDDEN_DIM), jnp.float32),
        )(
            gathered,
            ea_t,
            w1x_t,
            w1e_t,
            b1.reshape(1, HIDDEN_DIM),
            w2_t,
            b2.reshape(1, HIDDEN_DIM),
        )

    partials = []
    edge_base = 0
    for epw in PHASE_EPW:
        ep = epw * NW
        src_p = lax.dynamic_slice_in_dim(src, edge_base, ep)
        dst_p = lax.dynamic_slice_in_dim(dst, edge_base, ep)
        gathered = _make_sc_gather(epw)(x, src_p)
        messages = mlp(gathered, edge_base, ep)
        agg2 = _make_sc_scatter(epw)(messages, dst_p, zero)
        partials.append(agg2)
        edge_base += ep

    gru_in_specs = []
    gru_args = []
    for agg2 in partials:
        for half in range(NC):
            gru_in_specs.append(
                pl.BlockSpec(
                    (BN, HIDDEN_DIM),
                    lambda i, h=half: (h * (N_NODES // BN) + i, 0),
                )
            )
            gru_args.append(agg2)
    gru_in_specs += [
        pl.BlockSpec((BN, NODE_DIM), lambda i: (i, 0)),
        pl.BlockSpec((HIDDEN_DIM, 3 * NODE_DIM), lambda i: (0, 0)),
        pl.BlockSpec((NODE_DIM, 3 * NODE_DIM), lambda i: (0, 0)),
        pl.BlockSpec((1, 3 * NODE_DIM), lambda i: (0, 0)),
        pl.BlockSpec((1, 3 * NODE_DIM), lambda i: (0, 0)),
    ]
    gru_args += [
        x,
        W_ih.T,
        W_hh.T,
        b_ih.reshape(1, 3 * NODE_DIM),
        b_hh.reshape(1, 3 * NODE_DIM),
    ]

    x_new = pl.pallas_call(
        _gru_body,
        grid=(N_NODES // BN,),
        in_specs=gru_in_specs,
        out_specs=pl.BlockSpec((BN, NODE_DIM), lambda i: (i, 0)),
        out_shape=jax.ShapeDtypeStruct((N_NODES, NODE_DIM), jnp.float32),
    )(*gru_args)
    return x_new


# speedup vs baseline: 1.1397x; 1.1397x over previous
import functools

import jax
import jax.numpy as jnp
from jax import lax
from jax.experimental import pallas as pl
from jax.experimental.pallas import tpu as pltpu
from jax.experimental.pallas import tpu_sc as plsc

N_NODES = 10000
NODE_DIM = 128
EDGE_DIM = 16
HIDDEN_DIM = 128
N_EDGES = 320000

NC = 2
NS = 16
NW = NC * NS
PHASE_EPW = (5000, 5000)
P = len(PHASE_EPW)
CH = 80
STRIPE = 1000
KG = 6
KS = 4


def _split(epw, per_iter):
    outer = epw // per_iter
    rem = epw - outer * per_iter
    tail = []
    while rem >= CH:
        tail.append(CH)
        rem -= CH
    if rem:
        assert rem % 8 == 0
        tail.append(rem)
    return outer, tuple(tail)


@functools.cache
def _make_sc_gather(epw):
    outer, tail = _split(epw, KG * CH)
    mesh = plsc.VectorSubcoreMesh(core_axis_name="c", subcore_axis_name="s")

    def body_fn(x_hbm, src_hbm, out_hbm, idx_v, rows_v, sem):
        wid = lax.axis_index("s") * NC + lax.axis_index("c")

        def step(base, chunks):
            n = sum(chunks)
            pltpu.sync_copy(src_hbm.at[pl.ds(base, n)], idx_v.at[pl.ds(0, n)])
            cps, off = [], 0
            for c in chunks:
                cps.append(
                    pltpu.async_copy(
                        x_hbm.at[idx_v.at[pl.ds(off, c)]],
                        rows_v.at[pl.ds(off, c)],
                        sem,
                    )
                )
                off += c
            for cp in cps:
                cp.wait()
            pltpu.sync_copy(rows_v.at[pl.ds(0, n)], out_hbm.at[pl.ds(base, n)])

        def body(j, carry):
            step(wid * epw + j * (KG * CH), (CH,) * KG)
            return carry

        lax.fori_loop(0, outer, body, 0)
        if tail:
            step(wid * epw + outer * KG * CH, tail)

    return pl.kernel(
        body_fn,
        mesh=mesh,
        out_type=jax.ShapeDtypeStruct((epw * NW, NODE_DIM), jnp.float32),
        scratch_types=[
            pltpu.VMEM((KG * CH,), jnp.int32),
            pltpu.VMEM((KG * CH, NODE_DIM), jnp.float32),
            pltpu.SemaphoreType.DMA,
        ],
    )


@functools.cache
def _make_sc_scatter(epw):
    outer, tail = _split(epw, KS * CH)
    mesh = plsc.VectorSubcoreMesh(core_axis_name="c", subcore_axis_name="s")

    def body_fn(msg_hbm, dst_hbm, zero_hbm, out_hbm, idx_vs, idx_t, rows_v, agg_sh, isem):
        cid = lax.axis_index("c")
        sid = lax.axis_index("s")
        wid = sid * NC + cid
        @pl.when(sid < N_NODES // STRIPE)
        def _():
            pltpu.sync_copy(
                zero_hbm.at[pl.ds(sid * STRIPE, STRIPE)],
                agg_sh.at[pl.ds(sid * STRIPE, STRIPE)],
            )

        plsc.subcore_barrier()

        def step(base, bufs):
            n = sum(b.shape[0] for b in bufs)
            icps, off = [], 0
            for b in bufs:
                icps.append(
                    pltpu.async_copy(
                        dst_hbm.at[pl.ds(base + off, b.shape[0])], b, isem
                    )
                )
                off += b.shape[0]
            pltpu.sync_copy(msg_hbm.at[pl.ds(base, n)], rows_v.at[pl.ds(0, n)])
            for cp in icps:
                cp.wait()
            off = 0
            for b in bufs:
                pltpu.sync_copy(
                    rows_v.at[pl.ds(off, b.shape[0])], agg_sh.at[b], add=True
                )
                off += b.shape[0]

        def body(j, carry):
            step(wid * epw + j * (KS * CH), idx_vs)
            return carry

        lax.fori_loop(0, outer, body, 0)
        if tail:
            tail_bufs = [
                idx_vs[i] if c == CH else idx_t for i, c in enumerate(tail)
            ]
            step(wid * epw + outer * (KS * CH), tail_bufs)
        plsc.subcore_barrier()

        @pl.when(sid < N_NODES // STRIPE)
        def _():
            pltpu.sync_copy(
                agg_sh.at[pl.ds(sid * STRIPE, STRIPE)],
                out_hbm.at[pl.ds(cid * N_NODES + sid * STRIPE, STRIPE)],
            )

    return pl.kernel(
        body_fn,
        mesh=mesh,
        out_type=jax.ShapeDtypeStruct((NC * N_NODES, HIDDEN_DIM), jnp.float32),
        scratch_types=[
            [pltpu.VMEM((CH,), jnp.int32) for _ in range(KS)],
            pltpu.VMEM((40,), jnp.int32),
            pltpu.VMEM((KS * CH, HIDDEN_DIM), jnp.float32),
            pltpu.VMEM_SHARED((N_NODES, HIDDEN_DIM), jnp.float32),
            pltpu.SemaphoreType.DMA,
        ],
    )


def _mlp_body(gx_ref, ea_ref, w1x_ref, w1e_ref, b1_ref, w2_ref, b2_ref, out_ref):
    gx = gx_ref[...].astype(jnp.bfloat16)
    ea_t = ea_ref[...].astype(jnp.bfloat16)
    h = jnp.dot(gx, w1x_ref[...], preferred_element_type=jnp.float32)
    h = h + lax.dot_general(
        ea_t,
        w1e_ref[...],
        dimension_numbers=(((0,), (0,)), ((), ())),
        preferred_element_type=jnp.float32,
    )
    h = jnp.maximum(h + b1_ref[...], 0.0).astype(jnp.bfloat16)
    out_ref[...] = (
        jnp.dot(h, w2_ref[...], preferred_element_type=jnp.float32) + b2_ref[...]
    )


def _gru_body(*refs):
    part_refs = refs[: 2 * P]
    x_ref, wih_ref, whh_ref, bih_ref, bhh_ref, out_ref = refs[2 * P :]
    agg = part_refs[0][...]
    for r in part_refs[1:]:
        agg = agg + r[...]
    x = x_ref[...]
    gi = jnp.dot(agg, wih_ref[...], preferred_element_type=jnp.float32) + bih_ref[...]
    gh = jnp.dot(x, whh_ref[...], preferred_element_type=jnp.float32) + bhh_ref[...]
    i_r = gi[:, :NODE_DIM]
    i_z = gi[:, NODE_DIM : 2 * NODE_DIM]
    i_n = gi[:, 2 * NODE_DIM :]
    h_r = gh[:, :NODE_DIM]
    h_z = gh[:, NODE_DIM : 2 * NODE_DIM]
    h_n = gh[:, 2 * NODE_DIM :]
    r = jax.nn.sigmoid(i_r + h_r)
    z = jax.nn.sigmoid(i_z + h_z)
    n = jnp.tanh(i_n + r * h_n)
    out_ref[...] = (1.0 - z) * n + z * x


BE = 6400
BN = 2000


def kernel(x, edge_index, edge_attr, W1, b1, W2, b2, W_ih, b_ih, W_hh, b_hh):
    src = edge_index[0].astype(jnp.int32)
    dst = edge_index[1].astype(jnp.int32)
    ea_t = edge_attr.T

    w1x_t = W1[:, :NODE_DIM].T.astype(jnp.bfloat16)
    w1e_t = W1[:, NODE_DIM:].T.astype(jnp.bfloat16)
    w2_t = W2.T.astype(jnp.bfloat16)
    zero = jnp.zeros((N_NODES, HIDDEN_DIM), jnp.float32)

    def mlp(gathered, edge_base, n_edges):
        return pl.pallas_call(
            _mlp_body,
            grid=(n_edges // BE,),
            in_specs=[
                pl.BlockSpec((BE, NODE_DIM), lambda i: (i, 0)),
                pl.BlockSpec(
                    (EDGE_DIM, BE), lambda i, b=edge_base // BE: (0, b + i)
                ),
                pl.BlockSpec((NODE_DIM, HIDDEN_DIM), lambda i: (0, 0)),
                pl.BlockSpec((EDGE_DIM, HIDDEN_DIM), lambda i: (0, 0)),
                pl.BlockSpec((1, HIDDEN_DIM), lambda i: (0, 0)),
                pl.BlockSpec((HIDDEN_DIM, HIDDEN_DIM), lambda i: (0, 0)),
                pl.BlockSpec((1, HIDDEN_DIM), lambda i: (0, 0)),
            ],
            out_specs=pl.BlockSpec((BE, HIDDEN_DIM), lambda i: (i, 0)),
            out_shape=jax.ShapeDtypeStruct((n_edges, HIDDEN_DIM), jnp.float32),
        )(
            gathered,
            ea_t,
            w1x_t,
            w1e_t,
            b1.reshape(1, HIDDEN_DIM),
            w2_t,
            b2.reshape(1, HIDDEN_DIM),
        )

    partials = []
    edge_base = 0
    for epw in PHASE_EPW:
        ep = epw * NW
        src_p = lax.dynamic_slice_in_dim(src, edge_base, ep)
        dst_p = lax.dynamic_slice_in_dim(dst, edge_base, ep)
        gathered = _make_sc_gather(epw)(x, src_p)
        messages = mlp(gathered, edge_base, ep)
        agg2 = _make_sc_scatter(epw)(messages, dst_p, zero)
        partials.append(agg2)
        edge_base += ep

    gru_in_specs = []
    gru_args = []
    for agg2 in partials:
        for half in range(NC):
            gru_in_specs.append(
                pl.BlockSpec(
                    (BN, HIDDEN_DIM),
                    lambda i, h=half: (h * (N_NODES // BN) + i, 0),
                )
            )
            gru_args.append(agg2)
    gru_in_specs += [
        pl.BlockSpec((BN, NODE_DIM), lambda i: (i, 0)),
        pl.BlockSpec((HIDDEN_DIM, 3 * NODE_DIM), lambda i: (0, 0)),
        pl.BlockSpec((NODE_DIM, 3 * NODE_DIM), lambda i: (0, 0)),
        pl.BlockSpec((1, 3 * NODE_DIM), lambda i: (0, 0)),
        pl.BlockSpec((1, 3 * NODE_DIM), lambda i: (0, 0)),
    ]
    gru_args += [
        x,
        W_ih.T,
        W_hh.T,
        b_ih.reshape(1, 3 * NODE_DIM),
        b_hh.reshape(1, 3 * NODE_DIM),
    ]

    x_new = pl.pallas_call(
        _gru_body,
        grid=(N_NODES // BN,),
        in_specs=gru_in_specs,
        out_specs=pl.BlockSpec((BN, NODE_DIM), lambda i: (i, 0)),
        out_shape=jax.ShapeDtypeStruct((N_NODES, NODE_DIM), jnp.float32),
    )(*gru_args)
    return x_new


# speedup vs baseline: 1.1617x; 1.0193x over previous
import functools

import jax
import jax.numpy as jnp
from jax import lax
from jax.experimental import pallas as pl
from jax.experimental.pallas import tpu as pltpu
from jax.experimental.pallas import tpu_sc as plsc

N_NODES = 10000
NODE_DIM = 128
EDGE_DIM = 16
HIDDEN_DIM = 128
N_EDGES = 320000

NC = 2
NS = 16
NW = NC * NS
PHASE_EPW = (5000, 5000)
P = len(PHASE_EPW)
CH = 128
STRIPE = 1000
KG = 6
KS = 3


def _split(epw, per_iter):
    outer = epw // per_iter
    rem = epw - outer * per_iter
    tail = []
    while rem >= CH:
        tail.append(CH)
        rem -= CH
    if rem:
        assert rem % 8 == 0
        tail.append(rem)
    return outer, tuple(tail)


@functools.cache
def _make_sc_gather(epw):
    outer, tail = _split(epw, KG * CH)
    mesh = plsc.VectorSubcoreMesh(core_axis_name="c", subcore_axis_name="s")

    def body_fn(x_hbm, src_hbm, out_hbm, idx_v, rows_v, sem):
        wid = lax.axis_index("s") * NC + lax.axis_index("c")

        def step(base, chunks):
            n = sum(chunks)
            pltpu.sync_copy(src_hbm.at[pl.ds(base, n)], idx_v.at[pl.ds(0, n)])
            cps, off = [], 0
            for c in chunks:
                cps.append(
                    pltpu.async_copy(
                        x_hbm.at[idx_v.at[pl.ds(off, c)]],
                        rows_v.at[pl.ds(off, c)],
                        sem,
                    )
                )
                off += c
            for cp in cps:
                cp.wait()
            pltpu.sync_copy(rows_v.at[pl.ds(0, n)], out_hbm.at[pl.ds(base, n)])

        def body(j, carry):
            step(wid * epw + j * (KG * CH), (CH,) * KG)
            return carry

        lax.fori_loop(0, outer, body, 0)
        if tail:
            step(wid * epw + outer * KG * CH, tail)

    return pl.kernel(
        body_fn,
        mesh=mesh,
        out_type=jax.ShapeDtypeStruct((epw * NW, NODE_DIM), jnp.float32),
        scratch_types=[
            pltpu.VMEM((KG * CH,), jnp.int32),
            pltpu.VMEM((KG * CH, NODE_DIM), jnp.float32),
            pltpu.SemaphoreType.DMA,
        ],
    )


@functools.cache
def _make_sc_scatter(epw):
    outer, tail = _split(epw, KS * CH)
    mesh = plsc.VectorSubcoreMesh(core_axis_name="c", subcore_axis_name="s")

    def body_fn(msg_hbm, dst_hbm, zero_hbm, out_hbm, idx_vs, idx_t, rows_v, agg_sh, isem):
        cid = lax.axis_index("c")
        sid = lax.axis_index("s")
        wid = sid * NC + cid
        @pl.when(sid < N_NODES // STRIPE)
        def _():
            pltpu.sync_copy(
                zero_hbm.at[pl.ds(sid * STRIPE, STRIPE)],
                agg_sh.at[pl.ds(sid * STRIPE, STRIPE)],
            )

        plsc.subcore_barrier()

        def step(base, bufs):
            n = sum(b.shape[0] for b in bufs)
            icps, off = [], 0
            for b in bufs:
                icps.append(
                    pltpu.async_copy(
                        dst_hbm.at[pl.ds(base + off, b.shape[0])], b, isem
                    )
                )
                off += b.shape[0]
            pltpu.sync_copy(msg_hbm.at[pl.ds(base, n)], rows_v.at[pl.ds(0, n)])
            for cp in icps:
                cp.wait()
            off = 0
            for b in bufs:
                pltpu.sync_copy(
                    rows_v.at[pl.ds(off, b.shape[0])], agg_sh.at[b], add=True
                )
                off += b.shape[0]

        def body(j, carry):
            step(wid * epw + j * (KS * CH), idx_vs)
            return carry

        lax.fori_loop(0, outer, body, 0)
        if tail:
            step(wid * epw + outer * (KS * CH), idx_t)
        plsc.subcore_barrier()

        @pl.when(sid < N_NODES // STRIPE)
        def _():
            pltpu.sync_copy(
                agg_sh.at[pl.ds(sid * STRIPE, STRIPE)],
                out_hbm.at[pl.ds(cid * N_NODES + sid * STRIPE, STRIPE)],
            )

    return pl.kernel(
        body_fn,
        mesh=mesh,
        out_type=jax.ShapeDtypeStruct((NC * N_NODES, HIDDEN_DIM), jnp.float32),
        scratch_types=[
            [pltpu.VMEM((CH,), jnp.int32) for _ in range(KS)],
            [pltpu.VMEM((c,), jnp.int32) for c in tail],
            pltpu.VMEM((KS * CH, HIDDEN_DIM), jnp.float32),
            pltpu.VMEM_SHARED((N_NODES, HIDDEN_DIM), jnp.float32),
            pltpu.SemaphoreType.DMA,
        ],
    )


def _mlp_body(gx_ref, ea_ref, w1x_ref, w1e_ref, b1_ref, w2_ref, b2_ref, out_ref):
    gx = gx_ref[...].astype(jnp.bfloat16)
    ea_t = ea_ref[...].astype(jnp.bfloat16)
    h = jnp.dot(gx, w1x_ref[...], preferred_element_type=jnp.float32)
    h = h + lax.dot_general(
        ea_t,
        w1e_ref[...],
        dimension_numbers=(((0,), (0,)), ((), ())),
        preferred_element_type=jnp.float32,
    )
    h = jnp.maximum(h + b1_ref[...], 0.0).astype(jnp.bfloat16)
    out_ref[...] = (
        jnp.dot(h, w2_ref[...], preferred_element_type=jnp.float32) + b2_ref[...]
    )


def _gru_body(*refs):
    part_refs = refs[: 2 * P]
    x_ref, wih_ref, whh_ref, bih_ref, bhh_ref, out_ref = refs[2 * P :]
    agg = part_refs[0][...]
    for r in part_refs[1:]:
        agg = agg + r[...]
    x = x_ref[...]
    gi = jnp.dot(agg, wih_ref[...], preferred_element_type=jnp.float32) + bih_ref[...]
    gh = jnp.dot(x, whh_ref[...], preferred_element_type=jnp.float32) + bhh_ref[...]
    i_r = gi[:, :NODE_DIM]
    i_z = gi[:, NODE_DIM : 2 * NODE_DIM]
    i_n = gi[:, 2 * NODE_DIM :]
    h_r = gh[:, :NODE_DIM]
    h_z = gh[:, NODE_DIM : 2 * NODE_DIM]
    h_n = gh[:, 2 * NODE_DIM :]
    r = jax.nn.sigmoid(i_r + h_r)
    z = jax.nn.sigmoid(i_z + h_z)
    n = jnp.tanh(i_n + r * h_n)
    out_ref[...] = (1.0 - z) * n + z * x


BE = 6400
BN = 2000


def kernel(x, edge_index, edge_attr, W1, b1, W2, b2, W_ih, b_ih, W_hh, b_hh):
    src = edge_index[0].astype(jnp.int32)
    dst = edge_index[1].astype(jnp.int32)
    ea_t = edge_attr.T

    w1x_t = W1[:, :NODE_DIM].T.astype(jnp.bfloat16)
    w1e_t = W1[:, NODE_DIM:].T.astype(jnp.bfloat16)
    w2_t = W2.T.astype(jnp.bfloat16)
    zero = jnp.zeros((N_NODES, HIDDEN_DIM), jnp.float32)

    def mlp(gathered, edge_base, n_edges):
        return pl.pallas_call(
            _mlp_body,
            grid=(n_edges // BE,),
            in_specs=[
                pl.BlockSpec((BE, NODE_DIM), lambda i: (i, 0)),
                pl.BlockSpec(
                    (EDGE_DIM, BE), lambda i, b=edge_base // BE: (0, b + i)
                ),
                pl.BlockSpec((NODE_DIM, HIDDEN_DIM), lambda i: (0, 0)),
                pl.BlockSpec((EDGE_DIM, HIDDEN_DIM), lambda i: (0, 0)),
                pl.BlockSpec((1, HIDDEN_DIM), lambda i: (0, 0)),
                pl.BlockSpec((HIDDEN_DIM, HIDDEN_DIM), lambda i: (0, 0)),
                pl.BlockSpec((1, HIDDEN_DIM), lambda i: (0, 0)),
            ],
            out_specs=pl.BlockSpec((BE, HIDDEN_DIM), lambda i: (i, 0)),
            out_shape=jax.ShapeDtypeStruct((n_edges, HIDDEN_DIM), jnp.float32),
        )(
            gathered,
            ea_t,
            w1x_t,
            w1e_t,
            b1.reshape(1, HIDDEN_DIM),
            w2_t,
            b2.reshape(1, HIDDEN_DIM),
        )

    partials = []
    edge_base = 0
    for epw in PHASE_EPW:
        ep = epw * NW
        src_p = lax.dynamic_slice_in_dim(src, edge_base, ep)
        dst_p = lax.dynamic_slice_in_dim(dst, edge_base, ep)
        gathered = _make_sc_gather(epw)(x, src_p)
        messages = mlp(gathered, edge_base, ep)
        agg2 = _make_sc_scatter(epw)(messages, dst_p, zero)
        partials.append(agg2)
        edge_base += ep

    gru_in_specs = []
    gru_args = []
    for agg2 in partials:
        for half in range(NC):
            gru_in_specs.append(
                pl.BlockSpec(
                    (BN, HIDDEN_DIM),
                    lambda i, h=half: (h * (N_NODES // BN) + i, 0),
                )
            )
            gru_args.append(agg2)
    gru_in_specs += [
        pl.BlockSpec((BN, NODE_DIM), lambda i: (i, 0)),
        pl.BlockSpec((HIDDEN_DIM, 3 * NODE_DIM), lambda i: (0, 0)),
        pl.BlockSpec((NODE_DIM, 3 * NODE_DIM), lambda i: (0, 0)),
        pl.BlockSpec((1, 3 * NODE_DIM), lambda i: (0, 0)),
        pl.BlockSpec((1, 3 * NODE_DIM), lambda i: (0, 0)),
    ]
    gru_args += [
        x,
        W_ih.T,
        W_hh.T,
        b_ih.reshape(1, 3 * NODE_DIM),
        b_hh.reshape(1, 3 * NODE_DIM),
    ]

    x_new = pl.pallas_call(
        _gru_body,
        grid=(N_NODES // BN,),
        in_specs=gru_in_specs,
        out_specs=pl.BlockSpec((BN, NODE_DIM), lambda i: (i, 0)),
        out_shape=jax.ShapeDtypeStruct((N_NODES, NODE_DIM), jnp.float32),
    )(*gru_args)
    return x_new
